# baseline (speedup 1.0000x reference)
import functools

import jax
import jax.numpy as jnp
from jax import lax
from jax.experimental import pallas as pl
from jax.experimental.pallas import tpu as pltpu
from jax.experimental.pallas import tpu_sc as plsc

B = 16384
DIM = 64
NC = 2
NS = 16
NW = NC * NS
BPW = B // NW
CH = 128
NCH = BPW // CH
GRP = CH // 16

_ILV = plsc.PackFormat.INTERLEAVED


def _sc_body(h_hbm, r_hbm, t_hbm, ecat_hbm, rcat_hbm, out_hbm,
             hidx, ridx, tidx, hidx2, ridx2, tidx2,
             hrow, rrow, trow, tmp, outv, sem):
    wid = lax.axis_index("s") * NC + lax.axis_index("c")
    base = wid * BPW
    for c in range(NCH):
        off = base + c * CH
        pltpu.sync_copy(h_hbm.at[pl.ds(off, CH)], hidx)
        pltpu.sync_copy(r_hbm.at[pl.ds(off, CH)], ridx)
        pltpu.sync_copy(t_hbm.at[pl.ds(off, CH)], tidx)
        for v in range(CH // 16):
            sl = pl.ds(v * 16, 16)
            hidx2[sl] = lax.shift_right_logical(hidx[sl], 1)
            ridx2[sl] = lax.shift_right_logical(ridx[sl], 1)
            tidx2[sl] = lax.shift_right_logical(tidx[sl], 1)
        cps = [
            pltpu.async_copy(ecat_hbm.at[hidx2], hrow, sem),
            pltpu.async_copy(rcat_hbm.at[ridx2], rrow, sem),
            pltpu.async_copy(ecat_hbm.at[tidx2], trow, sem),
        ]
        for cp in cps:
            cp.wait()
        lanes = lax.broadcasted_iota(jnp.int32, (16,), 0)

        def group(g, _, c=c):
            gsl = pl.ds(g * 16, 16)
            hp = (hidx[gsl] & 1) * DIM
            rp = (ridx[gsl] & 1) * DIM
            tp = (tidx[gsl] & 1) * DIM
            for e in range(16):
                i = g * 16 + e
                ho = hp[e]
                ro = rp[e]
                to = tp[e]
                q = jnp.zeros((16,), jnp.float32)
                for k in range(DIM // 32):
                    re_off = k * 16
                    im_off = DIM // 2 + k * 16
                    a0, a1 = plsc.unpack(plsc.bitcast(
                        hrow[i, pl.ds(ho + re_off, 16)], jnp.bfloat16),
                        format=_ILV)
                    b0, b1 = plsc.unpack(plsc.bitcast(
                        hrow[i, pl.ds(ho + im_off, 16)], jnp.bfloat16),
                        format=_ILV)
                    c0, c1 = plsc.unpack(plsc.bitcast(
                        rrow[i, pl.ds(ro + re_off, 16)], jnp.bfloat16),
                        format=_ILV)
                    d0, d1 = plsc.unpack(plsc.bitcast(
                        rrow[i, pl.ds(ro + im_off, 16)], jnp.bfloat16),
                        format=_ILV)
                    e0, e1 = plsc.unpack(plsc.bitcast(
                        trow[i, pl.ds(to + re_off, 16)], jnp.bfloat16),
                        format=_ILV)
                    f0, f1 = plsc.unpack(plsc.bitcast(
                        trow[i, pl.ds(to + im_off, 16)], jnp.bfloat16),
                        format=_ILV)
                    q = q + e0 * (a0 * c0 - b0 * d0) + f0 * (a0 * d0 + b0 * c0)
                    q = q + e1 * (a1 * c1 - b1 * d1) + f1 * (a1 * d1 + b1 * c1)
                plsc.store_scatter(tmp, [lanes * 16 + e], q)
            s = tmp[pl.ds(0, 16)]
            for l in range(1, 16):
                s = s + tmp[pl.ds(l * 16, 16)]
            s = 1.0 / (1.0 + jnp.exp(-s))
            outv[pl.ds(c * CH + g * 16, 16)] = s
            return 0

        lax.fori_loop(0, GRP, group, 0)
    pltpu.sync_copy(outv, out_hbm.at[pl.ds(base, BPW)])


@jax.jit
def _run(h, r, t, ecat, rcat):
    mesh = plsc.VectorSubcoreMesh(core_axis_name="c", subcore_axis_name="s")
    idx_buf = pltpu.VMEM((CH,), jnp.int32)
    row_buf = pltpu.VMEM((CH, 2 * DIM), jnp.int32)
    kern = functools.partial(
        pl.kernel,
        mesh=mesh,
        compiler_params=pltpu.CompilerParams(needs_layout_passes=False),
        out_type=jax.ShapeDtypeStruct((B,), jnp.float32),
        scratch_types=[
            idx_buf, idx_buf, idx_buf, idx_buf, idx_buf, idx_buf,
            row_buf, row_buf, row_buf,
            pltpu.VMEM((256,), jnp.float32),
            pltpu.VMEM((BPW,), jnp.float32),
            pltpu.SemaphoreType.DMA,
        ],
    )(_sc_body)
    return kern(h, r, t, ecat, rcat)


def _pack_i32(re, im):
    cat = jnp.concatenate([re, im], axis=1).astype(jnp.bfloat16)
    pairs = cat.reshape(cat.shape[0] // 2, 2 * DIM, 2)
    return lax.bitcast_convert_type(pairs, jnp.int32)


def kernel(h, r, t, batch_size, emb_e_real, emb_e_img, emb_rel_real,
           emb_rel_img):
    ecat = _pack_i32(emb_e_real, emb_e_img)
    rcat = _pack_i32(emb_rel_real, emb_rel_img)
    score = _run(h, r, t, ecat, rcat)
    return score[:8192], score[8192:]


# speedup vs baseline: 46.3188x; 46.3188x over previous
import functools

import jax
import jax.numpy as jnp
from jax import lax
from jax.experimental import pallas as pl
from jax.experimental.pallas import tpu as pltpu
from jax.experimental.pallas import tpu_sc as plsc

B = 16384
DIM = 64
NC = 2
NS = 16
NW = NC * NS
BPW = B // NW
CH = 128
NCH = BPW // CH
GRP = CH // 16


def _sc_body(h_hbm, r_hbm, t_hbm, ecat_hbm, rcat_hbm, out_hbm,
             hidx, ridx, tidx, hrow, rrow, trow, tmp, outv, sem):
    wid = lax.axis_index("s") * NC + lax.axis_index("c")
    base = wid * BPW
    for c in range(NCH):
        off = base + c * CH
        pltpu.sync_copy(h_hbm.at[pl.ds(off, CH)], hidx)
        pltpu.sync_copy(r_hbm.at[pl.ds(off, CH)], ridx)
        pltpu.sync_copy(t_hbm.at[pl.ds(off, CH)], tidx)
        cps = [
            pltpu.async_copy(ecat_hbm.at[hidx], hrow, sem),
            pltpu.async_copy(rcat_hbm.at[ridx], rrow, sem),
            pltpu.async_copy(ecat_hbm.at[tidx], trow, sem),
        ]
        for cp in cps:
            cp.wait()
        lanes = lax.broadcasted_iota(jnp.int32, (16,), 0)

        def group(g, _, c=c):
            for e in range(16):
                i = g * 16 + e
                q = jnp.zeros((16,), jnp.float32)
                for k in range(DIM // 16):
                    re_sl = pl.ds(k * 16, 16)
                    im_sl = pl.ds(DIM + k * 16, 16)
                    a = hrow[i, re_sl]
                    b = hrow[i, im_sl]
                    cr = rrow[i, re_sl]
                    ci = rrow[i, im_sl]
                    dr = trow[i, re_sl]
                    di = trow[i, im_sl]
                    q = q + dr * (a * cr - b * ci) + di * (a * ci + b * cr)
                plsc.store_scatter(tmp, [lanes * 16 + e], q)
            s = tmp[pl.ds(0, 16)]
            for l in range(1, 16):
                s = s + tmp[pl.ds(l * 16, 16)]
            s = 1.0 / (1.0 + jnp.exp(-s))
            outv[pl.ds(c * CH + g * 16, 16)] = s
            return 0

        lax.fori_loop(0, GRP, group, 0)
    pltpu.sync_copy(outv, out_hbm.at[pl.ds(base, BPW)])


@jax.jit
def _run(h, r, t, ecat, rcat):
    mesh = plsc.VectorSubcoreMesh(core_axis_name="c", subcore_axis_name="s")
    gather_buf = pltpu.VMEM((CH, 2 * DIM), jnp.float32)
    kern = functools.partial(
        pl.kernel,
        mesh=mesh,
        compiler_params=pltpu.CompilerParams(needs_layout_passes=False),
        out_type=jax.ShapeDtypeStruct((B,), jnp.float32),
        scratch_types=[
            pltpu.VMEM((CH,), jnp.int32),
            pltpu.VMEM((CH,), jnp.int32),
            pltpu.VMEM((CH,), jnp.int32),
            gather_buf,
            gather_buf,
            gather_buf,
            pltpu.VMEM((256,), jnp.float32),
            pltpu.VMEM((BPW,), jnp.float32),
            pltpu.SemaphoreType.DMA,
        ],
    )(_sc_body)
    return kern(h, r, t, ecat, rcat)


def kernel(h, r, t, batch_size, emb_e_real, emb_e_img, emb_rel_real,
           emb_rel_img):
    ecat = jnp.concatenate([emb_e_real, emb_e_img], axis=1)
    rcat = jnp.concatenate([emb_rel_real, emb_rel_img], axis=1)
    score = _run(h, r, t, ecat, rcat)
    return score[:8192], score[8192:]
